# Initial kernel scaffold; baseline (speedup 1.0000x reference)
import functools

import jax
import jax.numpy as jnp
from jax import lax
from jax.experimental import pallas as pl
from jax.experimental.pallas import tpu as pltpu
from jax.experimental.pallas import tpu_sc as plsc

N = 10000
E = 320000
NC, NS, L = 2, 16, 16
NW = NC * NS
K = 128
NCH = 79
EPW = NCH * K
E_PAD = NW * EPW
TRASH = N
N_PAD = 10240
RPT = N_PAD // NS
ZR = 160
R_BLK = 1000
F32 = jnp.float32
HI = lax.Precision.HIGHEST


def _make_sc_scatter(C, with_count):
  mesh = plsc.VectorSubcoreMesh(core_axis_name="c", subcore_axis_name="s")
  out_type = [jax.ShapeDtypeStruct((NC * N_PAD, C), F32)]
  scratch = [
      pltpu.VMEM((NCH, K), jnp.int32),
      pltpu.VMEM((NCH, K), jnp.int32),
      pltpu.VMEM((K, C), F32),
      pltpu.VMEM((ZR, C), F32),
      pltpu.VMEM_SHARED((N_PAD, C), F32),
      pltpu.SemaphoreType.DMA,
  ]
  if with_count:
    out_type.append(jax.ShapeDtypeStruct((NC * N_PAD,), F32))
    scratch += [
        pltpu.VMEM((K,), F32),
        pltpu.VMEM((RPT,), F32),
        pltpu.VMEM_SHARED((N_PAD,), F32),
    ]

  def body(rows_hbm, cols_hbm, feat_hbm, out_hbm, *rest):
    if with_count:
      (cnt_hbm, rowb, colb, gbuf, zbuf, acc, gsem, ones, zcnt, cnt) = rest
    else:
      (rowb, colb, gbuf, zbuf, acc, gsem) = rest
    cid = lax.axis_index("c")
    sid = lax.axis_index("s")
    wid = sid * NC + cid
    base = sid * RPT

    pltpu.sync_copy(rows_hbm.at[wid], rowb)
    pltpu.sync_copy(cols_hbm.at[wid], colb)

    zero16 = jnp.zeros((L,), F32)

    def fix(j, carry):
      for i in range(K // L):
        r = rowb[j, pl.ds(i * L, L)]
        c = colb[j, pl.ds(i * L, L)]
        colb[j, pl.ds(i * L, L)] = jnp.where(r == c, TRASH, c)
      return carry
    lax.fori_loop(0, NCH, fix, 0)

    def zrow(j, carry):
      for i in range(C // L):
        zbuf[j, pl.ds(i * L, L)] = zero16
      return carry
    lax.fori_loop(0, ZR, zrow, 0)
    for p in range(RPT // ZR):
      pltpu.sync_copy(zbuf, acc.at[pl.ds(base + p * ZR, ZR)])

    if with_count:
      one16 = jnp.ones((L,), F32)
      for i in range(K // L):
        ones[pl.ds(i * L, L)] = one16
      def zc(j, carry):
        zcnt[pl.ds(j * L, L)] = zero16
        return carry
      lax.fori_loop(0, RPT // L, zc, 0)
      pltpu.sync_copy(zcnt, cnt.at[pl.ds(base, RPT)])

    plsc.subcore_barrier()

    def step(j, carry):
      pltpu.async_copy(feat_hbm.at[rowb.at[j]], gbuf, gsem).wait()
      pltpu.sync_copy(gbuf, acc.at[colb.at[j]], add=True)
      if with_count:
        pltpu.sync_copy(ones, cnt.at[colb.at[j]], add=True)
      return carry
    lax.fori_loop(0, NCH, step, 0)

    plsc.subcore_barrier()

    pltpu.sync_copy(acc.at[pl.ds(base, RPT)],
                    out_hbm.at[pl.ds(cid * N_PAD + base, RPT)])
    if with_count:
      pltpu.sync_copy(cnt.at[pl.ds(base, RPT)],
                      cnt_hbm.at[pl.ds(cid * N_PAD + base, RPT)])

  return pl.kernel(body, out_type=tuple(out_type), mesh=mesh,
                   scratch_types=scratch)


_sc_scatter_96 = _make_sc_scatter(96, True)
_sc_scatter_64 = _make_sc_scatter(64, False)


def _row_spec(Cdim):
  return pl.BlockSpec((R_BLK, Cdim), lambda i: (i, 0))


def _full_spec(shape):
  return pl.BlockSpec(shape, lambda i: tuple(0 for _ in shape))


def _mm(x, W):
  def body(xr, wr, outr):
    outr[...] = jnp.dot(xr[...], wr[...], precision=HI,
                        preferred_element_type=F32)
  return pl.pallas_call(
      body,
      grid=(N // R_BLK,),
      in_specs=[_row_spec(x.shape[1]), _full_spec(W.shape)],
      out_specs=_row_spec(W.shape[1]),
      out_shape=jax.ShapeDtypeStruct((N, W.shape[1]), F32),
  )(x, W)


def _combine1(s0, s1, c0, c1, hw1, x, W1r, b1, W2o):
  def body(s0r, s1r, c0r, c1r, hwr, xr, wrr, br, w2r,
           h1o, hw2o, dio, diso):
    deg = c0r[...] + c1r[...] + 1.0
    dinv = 1.0 / deg
    agg = (hwr[...] + s0r[...] + s1r[...]) * dinv
    root = jnp.dot(xr[...], wrr[...], precision=HI, preferred_element_type=F32)
    h1 = jnp.maximum(agg + br[...] + root, 0.0)
    h1o[...] = h1
    hw2o[...] = jnp.dot(h1, w2r[...], precision=HI, preferred_element_type=F32)
    dio[...] = dinv
    diso[...] = lax.rsqrt(deg)
  return pl.pallas_call(
      body,
      grid=(N // R_BLK,),
      in_specs=[_row_spec(96), _row_spec(96),
                _row_spec(1), _row_spec(1),
                _row_spec(96), _row_spec(128),
                _full_spec((128, 96)), _full_spec((1, 96)),
                _full_spec((96, 64))],
      out_specs=[_row_spec(96), _row_spec(64), _row_spec(1), _row_spec(1)],
      out_shape=[jax.ShapeDtypeStruct((N, 96), F32),
                 jax.ShapeDtypeStruct((N, 64), F32),
                 jax.ShapeDtypeStruct((N, 1), F32),
                 jax.ShapeDtypeStruct((N, 1), F32)],
  )(s0, s1, c0, c1, hw1, x, W1r, b1, W2o)


def _combine2(s0, s1, hw2, h1, W2r, b2, dinv, Wcat, dis):
  def body(s0r, s1r, hwr, h1r, wrr, br, dir_, wcr, disr, go):
    agg = (hwr[...] + s0r[...] + s1r[...]) * dir_[...]
    root = jnp.dot(h1r[...], wrr[...], precision=HI, preferred_element_type=F32)
    h2 = jnp.maximum(agg + br[...] + root, 0.0)
    go[...] = disr[...] * jnp.dot(h2, wcr[...], precision=HI,
                                  preferred_element_type=F32)
  return pl.pallas_call(
      body,
      grid=(N // R_BLK,),
      in_specs=[_row_spec(64), _row_spec(64), _row_spec(64), _row_spec(96),
                _full_spec((96, 64)), _full_spec((1, 64)),
                _row_spec(1), _full_spec((64, 64)), _row_spec(1)],
      out_specs=_row_spec(64),
      out_shape=jax.ShapeDtypeStruct((N, 64), F32),
  )(s0, s1, hw2, h1, W2r, b2, dinv, Wcat, dis)


def _combine3(s0, s1, g, dis, bcat):
  def body(s0r, s1r, gr, disr, br, outo):
    outo[...] = disr[...] * (gr[...] + s0r[...] + s1r[...]) + br[...]
  return pl.pallas_call(
      body,
      grid=(N // R_BLK,),
      in_specs=[_row_spec(64), _row_spec(64), _row_spec(64), _row_spec(1),
                _full_spec((1, 64))],
      out_specs=_row_spec(64),
      out_shape=jax.ShapeDtypeStruct((N, 64), F32),
  )(s0, s1, g, dis, bcat)


def kernel(x, edge_index, W1_out, b1, W1_root, W2_out, b2, W2_root,
           W_mu, b_mu, W_logstd, b_logstd):
  row = edge_index[0]
  col = edge_index[1]
  pad = E_PAD - E
  row = jnp.concatenate([row, jnp.zeros((pad,), jnp.int32)]).reshape(NW, NCH, K)
  col = jnp.concatenate([col, jnp.full((pad,), TRASH, jnp.int32)]
                        ).reshape(NW, NCH, K)

  Wcat = jnp.concatenate([W_mu, W_logstd], axis=1)
  bcat = jnp.concatenate([b_mu, b_logstd]).reshape(1, 64)

  hw1 = _mm(x, W1_out)
  scat1, cnt = _sc_scatter_96(row, col, hw1)
  c0 = cnt[:N_PAD].reshape(N_PAD, 1)
  c1 = cnt[N_PAD:].reshape(N_PAD, 1)
  h1, hw2, dinv, dis = _combine1(scat1[:N_PAD], scat1[N_PAD:], c0, c1,
                                 hw1, x, W1_root, b1.reshape(1, 96), W2_out)

  (scat2,) = _sc_scatter_64(row, col, hw2)
  g = _combine2(scat2[:N_PAD], scat2[N_PAD:], hw2, h1, W2_root,
                b2.reshape(1, 64), dinv, Wcat, dis)

  (scat3,) = _sc_scatter_64(row, col, g)
  out = _combine3(scat3[:N_PAD], scat3[N_PAD:], g, dis, bcat)
  return (out[:, :32], out[:, 32:])


# speedup vs baseline: 21.9411x; 21.9411x over previous
import functools

import jax
import jax.numpy as jnp
from jax import lax
from jax.experimental import pallas as pl
from jax.experimental.pallas import tpu as pltpu
from jax.experimental.pallas import tpu_sc as plsc

N = 10000
E = 320000
NC, NS, L = 2, 16, 16
NW = NC * NS
K = 128
NCH = 79
EPW = NCH * K
E_PAD = NW * EPW
TRASH = N
N_PAD = 10240
RPT = N_PAD // NS
ZR = 160
R_BLK = 1000
F32 = jnp.float32
HI = lax.Precision.HIGHEST


def _make_sc_scatter(C, with_count):
  mesh = plsc.VectorSubcoreMesh(core_axis_name="c", subcore_axis_name="s")
  out_type = [jax.ShapeDtypeStruct((NC * N_PAD, C), F32)]
  scratch = [
      pltpu.VMEM((NCH, K), jnp.int32),
      pltpu.VMEM((NCH, K), jnp.int32),
      pltpu.VMEM((K, C), F32),
      pltpu.VMEM((K, C), F32),
      pltpu.VMEM((ZR, C), F32),
      pltpu.VMEM_SHARED((N_PAD, C), F32),
      pltpu.SemaphoreType.DMA,
      pltpu.SemaphoreType.DMA,
  ]
  if with_count:
    out_type.append(jax.ShapeDtypeStruct((NC * N_PAD,), F32))
    scratch += [
        pltpu.VMEM((K,), F32),
        pltpu.VMEM((RPT,), F32),
        pltpu.VMEM_SHARED((N_PAD,), F32),
    ]

  def body(rows_hbm, cols_hbm, feat_hbm, out_hbm, *rest):
    if with_count:
      (cnt_hbm, rowb, colb, gbuf0, gbuf1, zbuf, acc, gsem0, gsem1,
       ones, zcnt, cnt) = rest
    else:
      (rowb, colb, gbuf0, gbuf1, zbuf, acc, gsem0, gsem1) = rest
    cid = lax.axis_index("c")
    sid = lax.axis_index("s")
    wid = sid * NC + cid
    base = sid * RPT

    pltpu.sync_copy(rows_hbm.at[wid], rowb)
    pltpu.sync_copy(cols_hbm.at[wid], colb)

    zero16 = jnp.zeros((L,), F32)

    def fix(j, carry):
      for i in range(K // L):
        r = rowb[j, pl.ds(i * L, L)]
        c = colb[j, pl.ds(i * L, L)]
        colb[j, pl.ds(i * L, L)] = jnp.where(r == c, TRASH, c)
      return carry
    lax.fori_loop(0, NCH, fix, 0)

    def zrow(j, carry):
      for i in range(C // L):
        zbuf[j, pl.ds(i * L, L)] = zero16
      return carry
    lax.fori_loop(0, ZR, zrow, 0)
    for p in range(RPT // ZR):
      pltpu.sync_copy(zbuf, acc.at[pl.ds(base + p * ZR, ZR)])

    if with_count:
      one16 = jnp.ones((L,), F32)
      for i in range(K // L):
        ones[pl.ds(i * L, L)] = one16
      def zc(j, carry):
        zcnt[pl.ds(j * L, L)] = zero16
        return carry
      lax.fori_loop(0, RPT // L, zc, 0)
      pltpu.sync_copy(zcnt, cnt.at[pl.ds(base, RPT)])

    plsc.subcore_barrier()

    def scat_add(buf, j):
      pltpu.sync_copy(buf, acc.at[colb.at[j]], add=True)
      if with_count:
        pltpu.sync_copy(ones, cnt.at[colb.at[j]], add=True)

    pltpu.async_copy(feat_hbm.at[rowb.at[0]], gbuf0, gsem0)
    def step(g, carry):
      j0 = 2 * g
      pltpu.make_async_copy(feat_hbm.at[rowb.at[j0]], gbuf0, gsem0).wait()
      pltpu.async_copy(feat_hbm.at[rowb.at[j0 + 1]], gbuf1, gsem1)
      scat_add(gbuf0, j0)
      pltpu.make_async_copy(feat_hbm.at[rowb.at[j0 + 1]], gbuf1, gsem1).wait()
      pltpu.async_copy(feat_hbm.at[rowb.at[j0 + 2]], gbuf0, gsem0)
      scat_add(gbuf1, j0 + 1)
      return carry
    lax.fori_loop(0, (NCH - 1) // 2, step, 0)
    pltpu.make_async_copy(feat_hbm.at[rowb.at[NCH - 1]], gbuf0, gsem0).wait()
    scat_add(gbuf0, NCH - 1)

    plsc.subcore_barrier()

    pltpu.sync_copy(acc.at[pl.ds(base, RPT)],
                    out_hbm.at[pl.ds(cid * N_PAD + base, RPT)])
    if with_count:
      pltpu.sync_copy(cnt.at[pl.ds(base, RPT)],
                      cnt_hbm.at[pl.ds(cid * N_PAD + base, RPT)])

  return pl.kernel(body, out_type=tuple(out_type), mesh=mesh,
                   scratch_types=scratch,
                   compiler_params=pltpu.CompilerParams(
                       use_tc_tiling_on_sc=False))


_sc_scatter_96 = _make_sc_scatter(96, True)
_sc_scatter_64 = _make_sc_scatter(64, False)


def _row_spec(Cdim):
  return pl.BlockSpec((R_BLK, Cdim), lambda i: (i, 0))


def _full_spec(shape):
  return pl.BlockSpec(shape, lambda i: tuple(0 for _ in shape))


def _mm(x, W):
  def body(xr, wr, outr):
    outr[...] = jnp.dot(xr[...], wr[...], precision=HI,
                        preferred_element_type=F32)
  return pl.pallas_call(
      body,
      grid=(N // R_BLK,),
      in_specs=[_row_spec(x.shape[1]), _full_spec(W.shape)],
      out_specs=_row_spec(W.shape[1]),
      out_shape=jax.ShapeDtypeStruct((N, W.shape[1]), F32),
  )(x, W)


def _combine1(s0, s1, c0, c1, hw1, x, W1r, b1, W2o):
  def body(s0r, s1r, c0r, c1r, hwr, xr, wrr, br, w2r,
           h1o, hw2o, dio, diso):
    deg = c0r[...] + c1r[...] + 1.0
    dinv = 1.0 / deg
    agg = (hwr[...] + s0r[...] + s1r[...]) * dinv
    root = jnp.dot(xr[...], wrr[...], precision=HI, preferred_element_type=F32)
    h1 = jnp.maximum(agg + br[...] + root, 0.0)
    h1o[...] = h1
    hw2o[...] = jnp.dot(h1, w2r[...], precision=HI, preferred_element_type=F32)
    dio[...] = dinv
    diso[...] = lax.rsqrt(deg)
  return pl.pallas_call(
      body,
      grid=(N // R_BLK,),
      in_specs=[_row_spec(96), _row_spec(96),
                _row_spec(1), _row_spec(1),
                _row_spec(96), _row_spec(128),
                _full_spec((128, 96)), _full_spec((1, 96)),
                _full_spec((96, 64))],
      out_specs=[_row_spec(96), _row_spec(64), _row_spec(1), _row_spec(1)],
      out_shape=[jax.ShapeDtypeStruct((N, 96), F32),
                 jax.ShapeDtypeStruct((N, 64), F32),
                 jax.ShapeDtypeStruct((N, 1), F32),
                 jax.ShapeDtypeStruct((N, 1), F32)],
  )(s0, s1, c0, c1, hw1, x, W1r, b1, W2o)


def _combine2(s0, s1, hw2, h1, W2r, b2, dinv, Wcat, dis):
  def body(s0r, s1r, hwr, h1r, wrr, br, dir_, wcr, disr, go):
    agg = (hwr[...] + s0r[...] + s1r[...]) * dir_[...]
    root = jnp.dot(h1r[...], wrr[...], precision=HI, preferred_element_type=F32)
    h2 = jnp.maximum(agg + br[...] + root, 0.0)
    go[...] = disr[...] * jnp.dot(h2, wcr[...], precision=HI,
                                  preferred_element_type=F32)
  return pl.pallas_call(
      body,
      grid=(N // R_BLK,),
      in_specs=[_row_spec(64), _row_spec(64), _row_spec(64), _row_spec(96),
                _full_spec((96, 64)), _full_spec((1, 64)),
                _row_spec(1), _full_spec((64, 64)), _row_spec(1)],
      out_specs=_row_spec(64),
      out_shape=jax.ShapeDtypeStruct((N, 64), F32),
  )(s0, s1, hw2, h1, W2r, b2, dinv, Wcat, dis)


def _combine3(s0, s1, g, dis, bcat):
  def body(s0r, s1r, gr, disr, br, outo):
    outo[...] = disr[...] * (gr[...] + s0r[...] + s1r[...]) + br[...]
  return pl.pallas_call(
      body,
      grid=(N // R_BLK,),
      in_specs=[_row_spec(64), _row_spec(64), _row_spec(64), _row_spec(1),
                _full_spec((1, 64))],
      out_specs=_row_spec(64),
      out_shape=jax.ShapeDtypeStruct((N, 64), F32),
  )(s0, s1, g, dis, bcat)


def kernel(x, edge_index, W1_out, b1, W1_root, W2_out, b2, W2_root,
           W_mu, b_mu, W_logstd, b_logstd):
  row = edge_index[0]
  col = edge_index[1]
  pad = E_PAD - E
  row = jnp.concatenate([row, jnp.zeros((pad,), jnp.int32)]).reshape(NW, NCH, K)
  col = jnp.concatenate([col, jnp.full((pad,), TRASH, jnp.int32)]
                        ).reshape(NW, NCH, K)

  Wcat = jnp.concatenate([W_mu, W_logstd], axis=1)
  bcat = jnp.concatenate([b_mu, b_logstd]).reshape(1, 64)

  hw1 = _mm(x, W1_out)
  scat1, cnt = _sc_scatter_96(row, col, hw1)
  c0 = cnt[:N_PAD].reshape(N_PAD, 1)
  c1 = cnt[N_PAD:].reshape(N_PAD, 1)
  h1, hw2, dinv, dis = _combine1(scat1[:N_PAD], scat1[N_PAD:], c0, c1,
                                 hw1, x, W1_root, b1.reshape(1, 96), W2_out)

  (scat2,) = _sc_scatter_64(row, col, hw2)
  g = _combine2(scat2[:N_PAD], scat2[N_PAD:], hw2, h1, W2_root,
                b2.reshape(1, 64), dinv, Wcat, dis)

  (scat3,) = _sc_scatter_64(row, col, g)
  out = _combine3(scat3[:N_PAD], scat3[N_PAD:], g, dis, bcat)
  return (out[:, :32], out[:, 32:])
